# baseline (speedup 1.0000x reference)
import jax
import jax.numpy as jnp
from jax import lax
from jax.experimental import pallas as pl
from jax.experimental.pallas import tpu as pltpu
from jax.experimental.pallas import tpu_sc as plsc

B = 16384
NN = 13
NC = 26
D = 32
V = 100000

NUM_CORES = 2
NUM_SUBCORES = 16
ICH = 4096
RCH = 4096
NCH = 4096
CPR = B // NCH
NKC = NN * CPR


def _body(emb_hbm, idx_hbm, xn_hbm, w_hbm, bias_hbm, out_hbm,
          plane_v, ich0_v, ich1_v, rch0_v, rch1_v, nch0_v, nch1_v,
          w_v, bias_v,
          sem_p, sem_i0, sem_i1, sem_o0, sem_o1,
          sem_x0, sem_x1, sem_n0, sem_n1):
    d = lax.axis_index("s") * NUM_CORES + lax.axis_index("c")
    ich = (ich0_v, ich1_v)
    rch = (rch0_v, rch1_v)
    nch = (nch0_v, nch1_v)
    sem_i = (sem_i0, sem_i1)
    sem_o = (sem_o0, sem_o1)
    sem_x = (sem_x0, sem_x1)
    sem_n = (sem_n0, sem_n1)

    pltpu.sync_copy(w_hbm, w_v)
    pltpu.sync_copy(bias_hbm, bias_v)
    d16 = jnp.full((16,), d, dtype=jnp.int32)

    def ich_src(f, h):
        return idx_hbm.at[pl.ds(f * B + h * ICH, ICH)]

    def xn_src(k):
        n = k // CPR
        c = k % CPR
        return xn_hbm.at[pl.ds(n * B + c * NCH, NCH)], n, c

    pltpu.async_copy(emb_hbm.at[0, d], plane_v, sem_p)
    pltpu.async_copy(ich_src(0, 0), ich[0], sem_i[0])
    src0, _, _ = xn_src(0)
    pltpu.async_copy(src0, nch[0], sem_x[0])

    def num_chunk(k, p):
        src, n, c = xn_src(k)
        wb = plsc.load_gather(w_v, [n * D + d16])
        bb = plsc.load_gather(bias_v, [n * D + d16])
        dst = out_hbm.at[n * D + d, pl.ds(c * NCH, NCH)]
        pltpu.make_async_copy(src, nch[p], sem_x[p]).wait()

        @plsc.parallel_loop(0, NCH, 16, unroll=8)
        def _(g):
            sl = pl.ds(g, 16)
            nch[p][sl] = nch[p][sl] * wb + bb

        pltpu.async_copy(nch[p], dst, sem_n[p])
        @pl.when(k >= 1)
        def _():
            pltpu.make_async_copy(nch[1 - p], dst, sem_n[1 - p]).wait()
        nsrc, _, _ = xn_src(jnp.minimum(k + 1, NKC - 1))
        pltpu.async_copy(nsrc, nch[1 - p], sem_x[1 - p])

    def cat_task(f, _):
        pltpu.make_async_copy(emb_hbm.at[f, d], plane_v, sem_p).wait()
        r = NN * D + f * D + d

        for h in range(B // ICH):
            pltpu.make_async_copy(ich_src(f, h), ich[h % 2], sem_i[h % 2]).wait()
            if h + 1 < B // ICH:
                nf, nh = f, h + 1
            else:
                nf, nh = jnp.minimum(f + 1, NC - 1), 0
            pltpu.async_copy(ich_src(nf, nh), ich[1 - h % 2], sem_i[1 - h % 2])

            for q in range(ICH // RCH):
                c = h * (ICH // RCH) + q
                p = c % 2
                b0 = c * RCH
                dst = out_hbm.at[r, pl.ds(b0, RCH)]
                if c >= 2:
                    pltpu.make_async_copy(rch[p], dst, sem_o[p]).wait()
                else:
                    @pl.when(f > 0)
                    def _():
                        pltpu.make_async_copy(rch[p], dst, sem_o[p]).wait()

                @plsc.parallel_loop(0, RCH, 16, unroll=8)
                def _(g):
                    rch[p][pl.ds(g, 16)] = plsc.load_gather(
                        plane_v, [ich[h % 2][pl.ds(q * RCH + g, 16)]])
                pltpu.async_copy(rch[p], dst, sem_o[p])

        pltpu.async_copy(emb_hbm.at[jnp.minimum(f + 1, NC - 1), d],
                         plane_v, sem_p)
        for j in range(NKC // NC):
            num_chunk(f * (NKC // NC) + j, j % 2)
        return 0

    lax.fori_loop(0, NC, cat_task, 0)

    last = NN * D + (NC - 1) * D + d
    nq = B // RCH
    pltpu.make_async_copy(emb_hbm.at[NC - 1, d], plane_v, sem_p).wait()
    pltpu.make_async_copy(ich_src(NC - 1, 0), ich[0], sem_i[0]).wait()
    pltpu.make_async_copy(
        rch[0], out_hbm.at[last, pl.ds((nq - 2) * RCH, RCH)], sem_o[0]
    ).wait()
    pltpu.make_async_copy(
        rch[1], out_hbm.at[last, pl.ds((nq - 1) * RCH, RCH)], sem_o[1]
    ).wait()
    lastn = (NN - 1) * D + d
    lsrc, _, _ = xn_src(NKC - 1)
    pltpu.make_async_copy(lsrc, nch[0], sem_x[0]).wait()
    pltpu.make_async_copy(
        nch[1], out_hbm.at[lastn, pl.ds((CPR - 1) * NCH, NCH)], sem_n[1]
    ).wait()


@jax.jit
def _run(emb_t, idx_t, xn_t, w1, bias1):
    mesh = plsc.VectorSubcoreMesh(
        core_axis_name="c", subcore_axis_name="s",
        num_cores=NUM_CORES, num_subcores=NUM_SUBCORES)
    f = pl.kernel(
        _body,
        out_type=jax.ShapeDtypeStruct(((NN + NC) * D, B), jnp.float32),
        mesh=mesh,
        scratch_types=[
            pltpu.VMEM((V,), jnp.float32),
            pltpu.VMEM((ICH,), jnp.int32),
            pltpu.VMEM((ICH,), jnp.int32),
            pltpu.VMEM((RCH,), jnp.float32),
            pltpu.VMEM((RCH,), jnp.float32),
            pltpu.VMEM((NCH,), jnp.float32),
            pltpu.VMEM((NCH,), jnp.float32),
            pltpu.VMEM((NN * D,), jnp.float32),
            pltpu.VMEM((NN * D,), jnp.float32),
        ] + [pltpu.SemaphoreType.DMA] * 9,
        compiler_params=pltpu.CompilerParams(needs_layout_passes=False),
    )
    return f(emb_t, idx_t, xn_t, w1, bias1)


def kernel(x_numeric, x_categorical, W_num, b_num, emb):
    emb_t = jnp.transpose(emb, (0, 2, 1))
    idx_t = jnp.transpose(x_categorical[:, :, 0], (1, 0)).reshape(NC * B)
    xn_t = jnp.transpose(x_numeric[:, :, 0], (1, 0)).reshape(NN * B)
    w1 = W_num.reshape(NN * D)
    bias1 = b_num.reshape(NN * D)
    out = _run(emb_t, idx_t, xn_t, w1, bias1)
    return jnp.transpose(out, (1, 0)).reshape(B, 1, (NN + NC) * D)


# speedup vs baseline: 1.1413x; 1.1413x over previous
import jax
import jax.numpy as jnp
from jax import lax
from jax.experimental import pallas as pl
from jax.experimental.pallas import tpu as pltpu
from jax.experimental.pallas import tpu_sc as plsc

B = 16384
NN = 13
NC = 26
D = 32
V = 100000

NUM_CORES = 2
NUM_SUBCORES = 16
ICH = 8192
RCH = 4096
NCH = 2048
CPR = B // NCH
NKC = NN * CPR


def _body(emb_hbm, idx_hbm, xn_hbm, w_hbm, bias_hbm, out_hbm,
          plane_v, ich0_v, ich1_v, rch0_v, rch1_v, nch0_v, nch1_v,
          w_v, bias_v,
          sem_p, sem_i0, sem_i1, sem_o0, sem_o1,
          sem_x0, sem_x1, sem_n0, sem_n1):
    d = lax.axis_index("s") * NUM_CORES + lax.axis_index("c")
    ich = (ich0_v, ich1_v)
    rch = (rch0_v, rch1_v)
    nch = (nch0_v, nch1_v)
    sem_i = (sem_i0, sem_i1)
    sem_o = (sem_o0, sem_o1)
    sem_x = (sem_x0, sem_x1)
    sem_n = (sem_n0, sem_n1)

    pltpu.sync_copy(w_hbm, w_v)
    pltpu.sync_copy(bias_hbm, bias_v)
    d16 = jnp.full((16,), d, dtype=jnp.int32)

    def ich_src(f, h):
        return idx_hbm.at[pl.ds(f * B + h * ICH, ICH)]

    def xn_src(k):
        n = k // CPR
        c = k % CPR
        return xn_hbm.at[pl.ds(n * B + c * NCH, NCH)], n, c

    pltpu.async_copy(emb_hbm.at[0, d], plane_v, sem_p)
    pltpu.async_copy(ich_src(0, 0), ich[0], sem_i[0])
    src0, _, _ = xn_src(0)
    pltpu.async_copy(src0, nch[0], sem_x[0])

    def num_chunk(k, p):
        src, n, c = xn_src(k)
        wb = plsc.load_gather(w_v, [n * D + d16])
        bb = plsc.load_gather(bias_v, [n * D + d16])
        dst = out_hbm.at[n * D + d, pl.ds(c * NCH, NCH)]
        pltpu.make_async_copy(src, nch[p], sem_x[p]).wait()

        @plsc.parallel_loop(0, NCH, 16, unroll=8)
        def _(g):
            sl = pl.ds(g, 16)
            nch[p][sl] = nch[p][sl] * wb + bb

        pltpu.async_copy(nch[p], dst, sem_n[p])
        @pl.when(k >= 1)
        def _():
            pltpu.make_async_copy(nch[1 - p], dst, sem_n[1 - p]).wait()
        nsrc, _, _ = xn_src(jnp.minimum(k + 1, NKC - 1))
        pltpu.async_copy(nsrc, nch[1 - p], sem_x[1 - p])

    def cat_task(f, _):
        pltpu.make_async_copy(emb_hbm.at[f, d], plane_v, sem_p).wait()
        r = NN * D + f * D + d

        for h in range(B // ICH):
            pltpu.make_async_copy(ich_src(f, h), ich[h % 2], sem_i[h % 2]).wait()
            if h + 1 < B // ICH:
                nf, nh = f, h + 1
            else:
                nf, nh = jnp.minimum(f + 1, NC - 1), 0
            pltpu.async_copy(ich_src(nf, nh), ich[1 - h % 2], sem_i[1 - h % 2])

            for q in range(ICH // RCH):
                c = h * (ICH // RCH) + q
                p = c % 2
                b0 = c * RCH
                dst = out_hbm.at[r, pl.ds(b0, RCH)]
                if c >= 2:
                    pltpu.make_async_copy(rch[p], dst, sem_o[p]).wait()
                else:
                    @pl.when(f > 0)
                    def _():
                        pltpu.make_async_copy(rch[p], dst, sem_o[p]).wait()

                @plsc.parallel_loop(0, RCH, 16, unroll=8)
                def _(g):
                    rch[p][pl.ds(g, 16)] = plsc.load_gather(
                        plane_v, [ich[h % 2][pl.ds(q * RCH + g, 16)]])
                pltpu.async_copy(rch[p], dst, sem_o[p])

        pltpu.async_copy(emb_hbm.at[jnp.minimum(f + 1, NC - 1), d],
                         plane_v, sem_p)
        for j in range(NKC // NC):
            num_chunk(f * (NKC // NC) + j, j % 2)
        return 0

    lax.fori_loop(0, NC, cat_task, 0)

    last = NN * D + (NC - 1) * D + d
    nq = B // RCH
    pltpu.make_async_copy(emb_hbm.at[NC - 1, d], plane_v, sem_p).wait()
    pltpu.make_async_copy(ich_src(NC - 1, 0), ich[0], sem_i[0]).wait()
    pltpu.make_async_copy(
        rch[0], out_hbm.at[last, pl.ds((nq - 2) * RCH, RCH)], sem_o[0]
    ).wait()
    pltpu.make_async_copy(
        rch[1], out_hbm.at[last, pl.ds((nq - 1) * RCH, RCH)], sem_o[1]
    ).wait()
    lastn = (NN - 1) * D + d
    lsrc, _, _ = xn_src(NKC - 1)
    pltpu.make_async_copy(lsrc, nch[0], sem_x[0]).wait()
    pltpu.make_async_copy(
        nch[1], out_hbm.at[lastn, pl.ds((CPR - 1) * NCH, NCH)], sem_n[1]
    ).wait()


@jax.jit
def _run(emb_t, idx_t, xn_t, w1, bias1):
    mesh = plsc.VectorSubcoreMesh(
        core_axis_name="c", subcore_axis_name="s",
        num_cores=NUM_CORES, num_subcores=NUM_SUBCORES)
    f = pl.kernel(
        _body,
        out_type=jax.ShapeDtypeStruct(((NN + NC) * D, B), jnp.float32),
        mesh=mesh,
        scratch_types=[
            pltpu.VMEM((V,), jnp.float32),
            pltpu.VMEM((ICH,), jnp.int32),
            pltpu.VMEM((ICH,), jnp.int32),
            pltpu.VMEM((RCH,), jnp.float32),
            pltpu.VMEM((RCH,), jnp.float32),
            pltpu.VMEM((NCH,), jnp.float32),
            pltpu.VMEM((NCH,), jnp.float32),
            pltpu.VMEM((NN * D,), jnp.float32),
            pltpu.VMEM((NN * D,), jnp.float32),
        ] + [pltpu.SemaphoreType.DMA] * 9,
        compiler_params=pltpu.CompilerParams(needs_layout_passes=False),
    )
    return f(emb_t, idx_t, xn_t, w1, bias1)


def kernel(x_numeric, x_categorical, W_num, b_num, emb):
    emb_t = jnp.transpose(emb, (0, 2, 1))
    idx_t = jnp.transpose(x_categorical[:, :, 0], (1, 0)).reshape(NC * B)
    xn_t = jnp.transpose(x_numeric[:, :, 0], (1, 0)).reshape(NN * B)
    w1 = W_num.reshape(NN * D)
    bias1 = b_num.reshape(NN * D)
    out = _run(emb_t, idx_t, xn_t, w1, bias1)
    return jnp.transpose(out, (1, 0)).reshape(B, 1, (NN + NC) * D)
